# baseline (speedup 1.0000x reference)
import functools

import jax
import jax.numpy as jnp
from jax import lax
from jax.experimental import pallas as pl
from jax.experimental.pallas import tpu as pltpu
from jax.experimental.pallas import tpu_sc as plsc

_NC = 2
_NS = 16
_NW = _NC * _NS
_CH = 64
_BLK = 1000


_NR = 4
_NI = 8


def _seg_sum_call(ps, src3, dst3, npad, split_cols):
    if split_cols:
        _, n, w = ps.shape
    else:
        n, w = ps.shape
    dt = ps.dtype
    lanes = 32 if dt == jnp.bfloat16 else 16
    _, nch, _ = src3.shape
    rpt = npad // _NS
    spt = n // _NS
    mesh = plsc.VectorSubcoreMesh(core_axis_name="c", subcore_axis_name="s")

    @functools.partial(
        pl.kernel,
        mesh=mesh,
        out_type=jax.ShapeDtypeStruct((_NC, npad, w), dt),
        scratch_types=(
            [pltpu.VMEM((_CH,), jnp.int32) for _ in range(2 * _NI)]
            + [pltpu.VMEM((_CH, w), dt) for _ in range(_NR)]
            + [
                pltpu.VMEM_SHARED((n, w), dt),
                pltpu.VMEM_SHARED((npad, w), dt),
            ]
            + [pltpu.SemaphoreType.DMA for _ in range(_NR + _NI)]
        ),
        compiler_params=pltpu.CompilerParams(use_tc_tiling_on_sc=False),
    )
    def k(p_hbm, src_hbm, dst_hbm, out_hbm, *scr):
        sidx = scr[:_NI]
        didx = scr[_NI : 2 * _NI]
        rows = scr[2 * _NI : 2 * _NI + _NR]
        stage = scr[2 * _NI + _NR]
        acc = scr[2 * _NI + _NR + 1]
        gs = scr[2 * _NI + _NR + 2 : 2 * _NI + _NR + 2 + _NR]
        fs = scr[2 * _NI + _NR + 2 + _NR :]
        cid = lax.axis_index("c")
        sid = lax.axis_index("s")
        wid = sid if split_cols else cid * _NS + sid
        zero = jnp.zeros((lanes,), dt)

        for q in range(_NI):
            pltpu.async_copy(src_hbm.at[wid, q], sidx[q], fs[q])
            pltpu.async_copy(dst_hbm.at[wid, q], didx[q], fs[q])

        if split_cols:
            pltpu.sync_copy(
                p_hbm.at[cid, pl.ds(sid * spt, spt)], stage.at[pl.ds(sid * spt, spt)]
            )
        else:
            pltpu.sync_copy(
                p_hbm.at[pl.ds(sid * spt, spt)], stage.at[pl.ds(sid * spt, spt)]
            )

        def zrow(r, carry):
            for c in range(w // lanes):
                rows[0][r, pl.ds(c * lanes, lanes)] = zero
            return carry

        lax.fori_loop(0, _CH, zrow, 0)
        for z in range(rpt // _CH):
            pltpu.sync_copy(rows[0], acc.at[pl.ds(sid * rpt + z * _CH, _CH)])
        plsc.subcore_barrier()
        for q in range(_NR - 1):
            pltpu.make_async_copy(src_hbm.at[wid, q], sidx[q], fs[q]).wait()
            pltpu.make_async_copy(dst_hbm.at[wid, q], didx[q], fs[q]).wait()
            pltpu.async_copy(stage.at[sidx[q]], rows[q], gs[q])

        def block(i, carry):
            for b in range(_NI):
                j = _NI * i + b
                r = b % _NR
                rn = (b + _NR - 1) % _NR
                fn = (b + _NR - 1) % _NI
                pltpu.make_async_copy(stage.at[sidx[b]], rows[r], gs[r]).wait()

                @pl.when(j + _NR - 1 < nch)
                def _():
                    pltpu.make_async_copy(
                        src_hbm.at[wid, 0], sidx[fn], fs[fn]
                    ).wait()
                    pltpu.make_async_copy(
                        dst_hbm.at[wid, 0], didx[fn], fs[fn]
                    ).wait()
                    pltpu.async_copy(stage.at[sidx[fn]], rows[rn], gs[rn])

                pltpu.sync_copy(rows[r], acc.at[didx[b]], add=True)

                @pl.when(j + _NI < nch)
                def _():
                    pltpu.async_copy(src_hbm.at[wid, j + _NI], sidx[b], fs[b])
                    pltpu.async_copy(dst_hbm.at[wid, j + _NI], didx[b], fs[b])
            return carry

        lax.fori_loop(0, nch // _NI, block, 0)
        plsc.subcore_barrier()
        pltpu.sync_copy(
            acc.at[pl.ds(sid * rpt, rpt)], out_hbm.at[cid, pl.ds(sid * rpt, rpt)]
        )

    return k(ps, src3, dst3)


def _deg_call(dst3, npad):
    w = 16
    _, nch, _ = dst3.shape
    rpt = npad // _NS
    mesh = plsc.VectorSubcoreMesh(core_axis_name="c", subcore_axis_name="s")

    @functools.partial(
        pl.kernel,
        mesh=mesh,
        out_type=jax.ShapeDtypeStruct((_NC, npad, w), jnp.float32),
        scratch_types=[
            pltpu.VMEM((nch, _CH), jnp.int32),
            pltpu.VMEM((_CH, w), jnp.float32),
            pltpu.VMEM((_CH, w), jnp.float32),
            pltpu.VMEM_SHARED((npad, w), jnp.float32),
            pltpu.SemaphoreType.DMA,
        ],
        compiler_params=pltpu.CompilerParams(use_tc_tiling_on_sc=False),
    )
    def k(dst_hbm, out_hbm, didx, ones_v, zrows, acc, sem):
        cid = lax.axis_index("c")
        sid = lax.axis_index("s")
        wid = cid * _NS + sid
        one = jnp.ones((16,), jnp.float32)
        zero = jnp.zeros((16,), jnp.float32)

        pltpu.sync_copy(dst_hbm.at[wid], didx)

        def fill(r, carry):
            ones_v[r, pl.ds(0, 16)] = one
            zrows[r, pl.ds(0, 16)] = zero
            return carry

        lax.fori_loop(0, _CH, fill, 0)
        for z in range(rpt // _CH):
            pltpu.sync_copy(zrows, acc.at[pl.ds(sid * rpt + z * _CH, _CH)])
        plsc.subcore_barrier()

        pltpu.async_copy(ones_v, acc.at[didx.at[0]], sem, add=True)

        def step(i, carry):
            pltpu.async_copy(ones_v, acc.at[didx.at[i + 1]], sem, add=True)
            pltpu.make_async_copy(ones_v, acc.at[didx.at[i]], sem).wait()
            return carry

        lax.fori_loop(0, nch - 1, step, 0)
        pltpu.make_async_copy(ones_v, acc.at[didx.at[nch - 1]], sem).wait()
        plsc.subcore_barrier()
        pltpu.sync_copy(
            acc.at[pl.ds(sid * rpt, rpt)], out_hbm.at[cid, pl.ds(sid * rpt, rpt)]
        )

    return k(dst3)


def _tc1_call(x, degp, w0r, w00, wfa, wf0, bfr):
    n, d = x.shape
    hid = w0r.shape[1]
    emb = wfa.shape[1]
    g = n // _BLK

    def body(x_ref, dp_ref, w0r_ref, w00_ref, wfa_ref, wf0_ref, bf_ref, p1_ref, oa_ref):
        deg = dp_ref[0, :, 0:1] + dp_ref[1, :, 0:1]
        xb = x_ref[...]
        p1_ref[...] = (jnp.dot(xb, w0r_ref[...]) + deg * w00_ref[...]).astype(
            jnp.bfloat16
        )
        oa_ref[...] = jnp.dot(xb, wfa_ref[...]) + deg * wf0_ref[...] + bf_ref[...]

    return pl.pallas_call(
        body,
        grid=(g,),
        in_specs=[
            pl.BlockSpec((_BLK, d), lambda i: (i, 0)),
            pl.BlockSpec((_NC, _BLK, 16), lambda i: (0, i, 0)),
            pl.BlockSpec((d, hid), lambda i: (0, 0)),
            pl.BlockSpec((1, hid), lambda i: (0, 0)),
            pl.BlockSpec((d, emb), lambda i: (0, 0)),
            pl.BlockSpec((1, emb), lambda i: (0, 0)),
            pl.BlockSpec((1, emb), lambda i: (0, 0)),
        ],
        out_specs=[
            pl.BlockSpec((_BLK, hid), lambda i: (i, 0)),
            pl.BlockSpec((_BLK, emb), lambda i: (i, 0)),
        ],
        out_shape=[
            jax.ShapeDtypeStruct((n, hid), jnp.bfloat16),
            jax.ShapeDtypeStruct((n, emb), jnp.float32),
        ],
    )(x, degp, w0r, w00, wfa, wf0, bfr)


def _tc2_call(s1p, degp, p1, b0r, w1, wfb, oa):
    n, hid = p1.shape
    emb = w1.shape[1]
    g = n // _BLK

    def body(s_ref, dp_ref, p1_ref, b0_ref, w1_ref, wfb_ref, oa_ref, p2_ref, o2_ref):
        deg = dp_ref[0, :, 0:1] + dp_ref[1, :, 0:1]
        degc = jnp.maximum(deg, 1.0)
        pos = deg > 0.0
        s = s_ref[0].astype(jnp.float32) + s_ref[1].astype(jnp.float32)
        agg = jnp.where(pos, s / degc, p1_ref[...].astype(jnp.float32))
        h2 = jnp.maximum(agg + b0_ref[...], 0.0)
        p2_ref[...] = jnp.dot(h2, w1_ref[...]).astype(jnp.bfloat16)
        o2_ref[...] = oa_ref[...] + jnp.dot(h2, wfb_ref[...])

    return pl.pallas_call(
        body,
        grid=(g,),
        in_specs=[
            pl.BlockSpec((_NC, _BLK, hid), lambda i: (0, i, 0)),
            pl.BlockSpec((_NC, _BLK, 16), lambda i: (0, i, 0)),
            pl.BlockSpec((_BLK, hid), lambda i: (i, 0)),
            pl.BlockSpec((1, hid), lambda i: (0, 0)),
            pl.BlockSpec((hid, emb), lambda i: (0, 0)),
            pl.BlockSpec((hid, emb), lambda i: (0, 0)),
            pl.BlockSpec((_BLK, emb), lambda i: (i, 0)),
        ],
        out_specs=[
            pl.BlockSpec((_BLK, emb), lambda i: (i, 0)),
            pl.BlockSpec((_BLK, emb), lambda i: (i, 0)),
        ],
        out_shape=[
            jax.ShapeDtypeStruct((n, emb), jnp.bfloat16),
            jax.ShapeDtypeStruct((n, emb), jnp.float32),
        ],
    )(s1p, degp, p1, b0r, w1, wfb, oa)


def _tc3_call(s2p, degp, p2, b1r, wfc, o2):
    n, emb = p2.shape
    g = n // _BLK

    def body(s_ref, dp_ref, p2_ref, b1_ref, wfc_ref, o2_ref, out_ref):
        deg = dp_ref[0, :, 0:1] + dp_ref[1, :, 0:1]
        s = s_ref[0].astype(jnp.float32) + s_ref[1].astype(jnp.float32)
        mean = s / jnp.maximum(deg, 1.0)
        agg = jnp.where(deg > 0.0, mean, p2_ref[...].astype(jnp.float32))
        h3 = jnp.maximum(agg + b1_ref[...], 0.0)
        out_ref[...] = o2_ref[...] + jnp.dot(h3, wfc_ref[...])

    return pl.pallas_call(
        body,
        grid=(g,),
        in_specs=[
            pl.BlockSpec((_NC, _BLK, emb), lambda i: (0, i, 0)),
            pl.BlockSpec((_NC, _BLK, 16), lambda i: (0, i, 0)),
            pl.BlockSpec((_BLK, emb), lambda i: (i, 0)),
            pl.BlockSpec((1, emb), lambda i: (0, 0)),
            pl.BlockSpec((emb, emb), lambda i: (0, 0)),
            pl.BlockSpec((_BLK, emb), lambda i: (i, 0)),
        ],
        out_specs=pl.BlockSpec((_BLK, emb), lambda i: (i, 0)),
        out_shape=jax.ShapeDtypeStruct((n, emb), jnp.float32),
    )(s2p, degp, p2, b1r, wfc, o2)


def kernel(x, edge_index, W0, b0, W1, b1, Wf, bf):
    n, d = x.shape
    e = edge_index.shape[1]
    hid = W0.shape[1]
    emb = W1.shape[1]

    grain = _NS * _CH
    npad = -(-(n + 1) // grain) * grain
    egrain = _NW * _CH * _NI
    epad = -(-e // egrain) * egrain
    nch = epad // (_NW * _CH)

    ept = epad // _NW
    ppt = ept - e // _NW
    spare = npad - n
    src = edge_index[0].reshape(_NW, e // _NW)
    dst = edge_index[1].reshape(_NW, e // _NW)
    pad_src = jnp.zeros((_NW, ppt), jnp.int32)
    pad_dst = (
        jnp.arange(_NW, dtype=jnp.int32)[:, None] * ppt
        + jnp.arange(ppt, dtype=jnp.int32)[None, :]
    ) % spare + n
    srcm = jnp.concatenate([src, pad_src], axis=1)
    dstm = jnp.concatenate([dst, pad_dst], axis=1)
    src3 = srcm.reshape(_NW, nch, _CH)
    dst3 = dstm.reshape(_NW, nch, _CH)
    src3h = srcm.reshape(_NS, 2 * nch, _CH)
    dst3h = dstm.reshape(_NS, 2 * nch, _CH)

    degp = _deg_call(dst3, npad)

    p1, oa = _tc1_call(
        x,
        degp,
        W0[1:],
        W0[0:1],
        Wf[1 : d + 1],
        Wf[0:1],
        bf.reshape(1, emb),
    )

    s1p = _seg_sum_call(p1, src3, dst3, npad, split_cols=False)
    p2, o2 = _tc2_call(
        s1p, degp, p1, b0.reshape(1, hid), W1, Wf[d + 1 : d + 1 + hid], oa
    )

    s2p = _seg_sum_call(p2, src3, dst3, npad, split_cols=False)
    out = _tc3_call(s2p, degp, p2, b1.reshape(1, emb), Wf[d + 1 + hid :], o2)
    return out


# speedup vs baseline: 1.0272x; 1.0272x over previous
import functools

import jax
import jax.numpy as jnp
from jax import lax
from jax.experimental import pallas as pl
from jax.experimental.pallas import tpu as pltpu
from jax.experimental.pallas import tpu_sc as plsc

_NC = 2
_NS = 16
_NW = _NC * _NS
_CH = 64
_BLK = 2000


_NR = 4
_NI = 8


def _seg_sum_call(ps, src3, dst3, npad, split_cols):
    if split_cols:
        _, n, w = ps.shape
    else:
        n, w = ps.shape
    dt = ps.dtype
    lanes = 32 if dt == jnp.bfloat16 else 16
    _, nch, _ = src3.shape
    rpt = npad // _NS
    spt = n // _NS
    mesh = plsc.VectorSubcoreMesh(core_axis_name="c", subcore_axis_name="s")

    @functools.partial(
        pl.kernel,
        mesh=mesh,
        out_type=jax.ShapeDtypeStruct((_NC, npad, w), dt),
        scratch_types=(
            [pltpu.VMEM((_CH,), jnp.int32) for _ in range(2 * _NI)]
            + [pltpu.VMEM((_CH, w), dt) for _ in range(_NR)]
            + [
                pltpu.VMEM_SHARED((n, w), dt),
                pltpu.VMEM_SHARED((npad, w), dt),
            ]
            + [pltpu.SemaphoreType.DMA for _ in range(_NR + _NI)]
        ),
        compiler_params=pltpu.CompilerParams(use_tc_tiling_on_sc=False),
    )
    def k(p_hbm, src_hbm, dst_hbm, out_hbm, *scr):
        sidx = scr[:_NI]
        didx = scr[_NI : 2 * _NI]
        rows = scr[2 * _NI : 2 * _NI + _NR]
        stage = scr[2 * _NI + _NR]
        acc = scr[2 * _NI + _NR + 1]
        gs = scr[2 * _NI + _NR + 2 : 2 * _NI + _NR + 2 + _NR]
        fs = scr[2 * _NI + _NR + 2 + _NR :]
        cid = lax.axis_index("c")
        sid = lax.axis_index("s")
        wid = sid if split_cols else cid * _NS + sid
        zero = jnp.zeros((lanes,), dt)

        for q in range(_NI):
            pltpu.async_copy(src_hbm.at[wid, q], sidx[q], fs[q])
            pltpu.async_copy(dst_hbm.at[wid, q], didx[q], fs[q])

        if split_cols:
            pltpu.sync_copy(
                p_hbm.at[cid, pl.ds(sid * spt, spt)], stage.at[pl.ds(sid * spt, spt)]
            )
        else:
            pltpu.sync_copy(
                p_hbm.at[pl.ds(sid * spt, spt)], stage.at[pl.ds(sid * spt, spt)]
            )

        def zrow(r, carry):
            for c in range(w // lanes):
                rows[0][r, pl.ds(c * lanes, lanes)] = zero
            return carry

        lax.fori_loop(0, _CH, zrow, 0)
        for z in range(rpt // _CH):
            pltpu.sync_copy(rows[0], acc.at[pl.ds(sid * rpt + z * _CH, _CH)])
        plsc.subcore_barrier()
        for q in range(_NR - 1):
            pltpu.make_async_copy(src_hbm.at[wid, q], sidx[q], fs[q]).wait()
            pltpu.make_async_copy(dst_hbm.at[wid, q], didx[q], fs[q]).wait()
            pltpu.async_copy(stage.at[sidx[q]], rows[q], gs[q])

        def block(i, carry):
            for b in range(_NI):
                j = _NI * i + b
                r = b % _NR
                rn = (b + _NR - 1) % _NR
                fn = (b + _NR - 1) % _NI
                pltpu.make_async_copy(stage.at[sidx[b]], rows[r], gs[r]).wait()

                @pl.when(j + _NR - 1 < nch)
                def _():
                    pltpu.make_async_copy(
                        src_hbm.at[wid, 0], sidx[fn], fs[fn]
                    ).wait()
                    pltpu.make_async_copy(
                        dst_hbm.at[wid, 0], didx[fn], fs[fn]
                    ).wait()
                    pltpu.async_copy(stage.at[sidx[fn]], rows[rn], gs[rn])

                pltpu.sync_copy(rows[r], acc.at[didx[b]], add=True)

                @pl.when(j + _NI < nch)
                def _():
                    pltpu.async_copy(src_hbm.at[wid, j + _NI], sidx[b], fs[b])
                    pltpu.async_copy(dst_hbm.at[wid, j + _NI], didx[b], fs[b])
            return carry

        lax.fori_loop(0, nch // _NI, block, 0)
        plsc.subcore_barrier()
        pltpu.sync_copy(
            acc.at[pl.ds(sid * rpt, rpt)], out_hbm.at[cid, pl.ds(sid * rpt, rpt)]
        )

    return k(ps, src3, dst3)


def _deg_call(dst3, npad):
    w = 16
    _, nch, _ = dst3.shape
    rpt = npad // _NS
    mesh = plsc.VectorSubcoreMesh(core_axis_name="c", subcore_axis_name="s")

    @functools.partial(
        pl.kernel,
        mesh=mesh,
        out_type=jax.ShapeDtypeStruct((_NC, npad, w), jnp.float32),
        scratch_types=[
            pltpu.VMEM((nch, _CH), jnp.int32),
            pltpu.VMEM((_CH, w), jnp.float32),
            pltpu.VMEM((_CH, w), jnp.float32),
            pltpu.VMEM_SHARED((npad, w), jnp.float32),
            pltpu.SemaphoreType.DMA,
        ],
        compiler_params=pltpu.CompilerParams(use_tc_tiling_on_sc=False),
    )
    def k(dst_hbm, out_hbm, didx, ones_v, zrows, acc, sem):
        cid = lax.axis_index("c")
        sid = lax.axis_index("s")
        wid = cid * _NS + sid
        one = jnp.ones((16,), jnp.float32)
        zero = jnp.zeros((16,), jnp.float32)

        pltpu.sync_copy(dst_hbm.at[wid], didx)

        def fill(r, carry):
            ones_v[r, pl.ds(0, 16)] = one
            zrows[r, pl.ds(0, 16)] = zero
            return carry

        lax.fori_loop(0, _CH, fill, 0)
        for z in range(rpt // _CH):
            pltpu.sync_copy(zrows, acc.at[pl.ds(sid * rpt + z * _CH, _CH)])
        plsc.subcore_barrier()

        pltpu.async_copy(ones_v, acc.at[didx.at[0]], sem, add=True)

        def step(i, carry):
            pltpu.async_copy(ones_v, acc.at[didx.at[i + 1]], sem, add=True)
            pltpu.make_async_copy(ones_v, acc.at[didx.at[i]], sem).wait()
            return carry

        lax.fori_loop(0, nch - 1, step, 0)
        pltpu.make_async_copy(ones_v, acc.at[didx.at[nch - 1]], sem).wait()
        plsc.subcore_barrier()
        pltpu.sync_copy(
            acc.at[pl.ds(sid * rpt, rpt)], out_hbm.at[cid, pl.ds(sid * rpt, rpt)]
        )

    return k(dst3)


def _tc1_call(x, degp, w0r, w00, wfa, wf0, bfr):
    n, d = x.shape
    hid = w0r.shape[1]
    emb = wfa.shape[1]
    g = n // _BLK

    def body(x_ref, dp_ref, w0r_ref, w00_ref, wfa_ref, wf0_ref, bf_ref, p1_ref, oa_ref):
        deg = dp_ref[0, :, 0:1] + dp_ref[1, :, 0:1]
        xb = x_ref[...]
        p1_ref[...] = (jnp.dot(xb, w0r_ref[...]) + deg * w00_ref[...]).astype(
            jnp.bfloat16
        )
        oa_ref[...] = jnp.dot(xb, wfa_ref[...]) + deg * wf0_ref[...] + bf_ref[...]

    return pl.pallas_call(
        body,
        grid=(g,),
        in_specs=[
            pl.BlockSpec((_BLK, d), lambda i: (i, 0)),
            pl.BlockSpec((_NC, _BLK, 16), lambda i: (0, i, 0)),
            pl.BlockSpec((d, hid), lambda i: (0, 0)),
            pl.BlockSpec((1, hid), lambda i: (0, 0)),
            pl.BlockSpec((d, emb), lambda i: (0, 0)),
            pl.BlockSpec((1, emb), lambda i: (0, 0)),
            pl.BlockSpec((1, emb), lambda i: (0, 0)),
        ],
        out_specs=[
            pl.BlockSpec((_BLK, hid), lambda i: (i, 0)),
            pl.BlockSpec((_BLK, emb), lambda i: (i, 0)),
        ],
        out_shape=[
            jax.ShapeDtypeStruct((n, hid), jnp.bfloat16),
            jax.ShapeDtypeStruct((n, emb), jnp.float32),
        ],
    )(x, degp, w0r, w00, wfa, wf0, bfr)


def _tc2_call(s1p, degp, p1, b0r, w1, wfb, oa):
    n, hid = p1.shape
    emb = w1.shape[1]
    g = n // _BLK

    def body(s_ref, dp_ref, p1_ref, b0_ref, w1_ref, wfb_ref, oa_ref, p2_ref, o2_ref):
        deg = dp_ref[0, :, 0:1] + dp_ref[1, :, 0:1]
        degc = jnp.maximum(deg, 1.0)
        pos = deg > 0.0
        s = s_ref[0].astype(jnp.float32) + s_ref[1].astype(jnp.float32)
        agg = jnp.where(pos, s / degc, p1_ref[...].astype(jnp.float32))
        h2 = jnp.maximum(agg + b0_ref[...], 0.0)
        p2_ref[...] = jnp.dot(h2, w1_ref[...]).astype(jnp.bfloat16)
        o2_ref[...] = oa_ref[...] + jnp.dot(h2, wfb_ref[...])

    return pl.pallas_call(
        body,
        grid=(g,),
        in_specs=[
            pl.BlockSpec((_NC, _BLK, hid), lambda i: (0, i, 0)),
            pl.BlockSpec((_NC, _BLK, 16), lambda i: (0, i, 0)),
            pl.BlockSpec((_BLK, hid), lambda i: (i, 0)),
            pl.BlockSpec((1, hid), lambda i: (0, 0)),
            pl.BlockSpec((hid, emb), lambda i: (0, 0)),
            pl.BlockSpec((hid, emb), lambda i: (0, 0)),
            pl.BlockSpec((_BLK, emb), lambda i: (i, 0)),
        ],
        out_specs=[
            pl.BlockSpec((_BLK, emb), lambda i: (i, 0)),
            pl.BlockSpec((_BLK, emb), lambda i: (i, 0)),
        ],
        out_shape=[
            jax.ShapeDtypeStruct((n, emb), jnp.bfloat16),
            jax.ShapeDtypeStruct((n, emb), jnp.float32),
        ],
    )(s1p, degp, p1, b0r, w1, wfb, oa)


def _tc3_call(s2p, degp, p2, b1r, wfc, o2):
    n, emb = p2.shape
    g = n // _BLK

    def body(s_ref, dp_ref, p2_ref, b1_ref, wfc_ref, o2_ref, out_ref):
        deg = dp_ref[0, :, 0:1] + dp_ref[1, :, 0:1]
        s = s_ref[0].astype(jnp.float32) + s_ref[1].astype(jnp.float32)
        mean = s / jnp.maximum(deg, 1.0)
        agg = jnp.where(deg > 0.0, mean, p2_ref[...].astype(jnp.float32))
        h3 = jnp.maximum(agg + b1_ref[...], 0.0)
        out_ref[...] = o2_ref[...] + jnp.dot(h3, wfc_ref[...])

    return pl.pallas_call(
        body,
        grid=(g,),
        in_specs=[
            pl.BlockSpec((_NC, _BLK, emb), lambda i: (0, i, 0)),
            pl.BlockSpec((_NC, _BLK, 16), lambda i: (0, i, 0)),
            pl.BlockSpec((_BLK, emb), lambda i: (i, 0)),
            pl.BlockSpec((1, emb), lambda i: (0, 0)),
            pl.BlockSpec((emb, emb), lambda i: (0, 0)),
            pl.BlockSpec((_BLK, emb), lambda i: (i, 0)),
        ],
        out_specs=pl.BlockSpec((_BLK, emb), lambda i: (i, 0)),
        out_shape=jax.ShapeDtypeStruct((n, emb), jnp.float32),
    )(s2p, degp, p2, b1r, wfc, o2)


def kernel(x, edge_index, W0, b0, W1, b1, Wf, bf):
    n, d = x.shape
    e = edge_index.shape[1]
    hid = W0.shape[1]
    emb = W1.shape[1]

    grain = _NS * _CH
    npad = -(-(n + 1) // grain) * grain
    egrain = _NW * _CH * _NI
    epad = -(-e // egrain) * egrain
    nch = epad // (_NW * _CH)

    ept = epad // _NW
    ppt = ept - e // _NW
    spare = npad - n
    src = edge_index[0].reshape(_NW, e // _NW)
    dst = edge_index[1].reshape(_NW, e // _NW)
    pad_src = jnp.zeros((_NW, ppt), jnp.int32)
    pad_dst = (
        jnp.arange(_NW, dtype=jnp.int32)[:, None] * ppt
        + jnp.arange(ppt, dtype=jnp.int32)[None, :]
    ) % spare + n
    srcm = jnp.concatenate([src, pad_src], axis=1)
    dstm = jnp.concatenate([dst, pad_dst], axis=1)
    src3 = srcm.reshape(_NW, nch, _CH)
    dst3 = dstm.reshape(_NW, nch, _CH)
    src3h = srcm.reshape(_NS, 2 * nch, _CH)
    dst3h = dstm.reshape(_NS, 2 * nch, _CH)

    degp = _deg_call(dst3, npad)

    p1, oa = _tc1_call(
        x,
        degp,
        W0[1:],
        W0[0:1],
        Wf[1 : d + 1],
        Wf[0:1],
        bf.reshape(1, emb),
    )

    s1p = _seg_sum_call(p1, src3, dst3, npad, split_cols=False)
    p2, o2 = _tc2_call(
        s1p, degp, p1, b0.reshape(1, hid), W1, Wf[d + 1 : d + 1 + hid], oa
    )

    s2p = _seg_sum_call(p2, src3, dst3, npad, split_cols=False)
    out = _tc3_call(s2p, degp, p2, b1.reshape(1, emb), Wf[d + 1 + hid :], o2)
    return out


# speedup vs baseline: 1.0386x; 1.0111x over previous
import functools

import jax
import jax.numpy as jnp
from jax import lax
from jax.experimental import pallas as pl
from jax.experimental.pallas import tpu as pltpu
from jax.experimental.pallas import tpu_sc as plsc

_NC = 2
_NS = 16
_NW = _NC * _NS
_CH = 64
_BLK = 2000


_NR = 4
_NI = 8


def _seg_sum_call(ps, src3, dst3, npad, split_cols):
    if split_cols:
        _, n, w = ps.shape
    else:
        n, w = ps.shape
    dt = ps.dtype
    lanes = 32 if dt == jnp.bfloat16 else 16
    _, nch, _ = src3.shape
    rpt = npad // _NS
    spt = n // _NS
    mesh = plsc.VectorSubcoreMesh(core_axis_name="c", subcore_axis_name="s")

    @functools.partial(
        pl.kernel,
        mesh=mesh,
        out_type=jax.ShapeDtypeStruct((_NC, npad, w), dt),
        scratch_types=(
            [pltpu.VMEM((_CH,), jnp.int32) for _ in range(2 * _NI)]
            + [pltpu.VMEM((_CH, w), dt) for _ in range(_NR)]
            + [
                pltpu.VMEM_SHARED((n, w), dt),
                pltpu.VMEM_SHARED((npad, w), dt),
            ]
            + [pltpu.SemaphoreType.DMA for _ in range(_NR + _NI)]
        ),
        compiler_params=pltpu.CompilerParams(use_tc_tiling_on_sc=False),
    )
    def k(p_hbm, src_hbm, dst_hbm, out_hbm, *scr):
        sidx = scr[:_NI]
        didx = scr[_NI : 2 * _NI]
        rows = scr[2 * _NI : 2 * _NI + _NR]
        stage = scr[2 * _NI + _NR]
        acc = scr[2 * _NI + _NR + 1]
        gs = scr[2 * _NI + _NR + 2 : 2 * _NI + _NR + 2 + _NR]
        fs = scr[2 * _NI + _NR + 2 + _NR :]
        cid = lax.axis_index("c")
        sid = lax.axis_index("s")
        wid = sid if split_cols else cid * _NS + sid
        zero = jnp.zeros((lanes,), dt)

        for q in range(_NI):
            pltpu.async_copy(src_hbm.at[wid, q], sidx[q], fs[q])
            pltpu.async_copy(dst_hbm.at[wid, q], didx[q], fs[q])

        if split_cols:
            pltpu.sync_copy(
                p_hbm.at[cid, pl.ds(sid * spt, spt)], stage.at[pl.ds(sid * spt, spt)]
            )
        else:
            pltpu.sync_copy(
                p_hbm.at[pl.ds(sid * spt, spt)], stage.at[pl.ds(sid * spt, spt)]
            )

        def zrow(r, carry):
            for c in range(w // lanes):
                rows[0][r, pl.ds(c * lanes, lanes)] = zero
            return carry

        lax.fori_loop(0, _CH, zrow, 0)
        for z in range(rpt // _CH):
            pltpu.sync_copy(rows[0], acc.at[pl.ds(sid * rpt + z * _CH, _CH)])
        plsc.subcore_barrier()
        for q in range(_NR - 1):
            pltpu.make_async_copy(src_hbm.at[wid, q], sidx[q], fs[q]).wait()
            pltpu.make_async_copy(dst_hbm.at[wid, q], didx[q], fs[q]).wait()
            pltpu.async_copy(stage.at[sidx[q]], rows[q], gs[q])

        def block(i, carry):
            for b in range(_NI):
                j = _NI * i + b
                r = b % _NR
                rn = (b + _NR - 1) % _NR
                fn = (b + _NR - 1) % _NI
                pltpu.make_async_copy(stage.at[sidx[b]], rows[r], gs[r]).wait()

                @pl.when(j + _NR - 1 < nch)
                def _():
                    pltpu.make_async_copy(
                        src_hbm.at[wid, 0], sidx[fn], fs[fn]
                    ).wait()
                    pltpu.make_async_copy(
                        dst_hbm.at[wid, 0], didx[fn], fs[fn]
                    ).wait()
                    pltpu.async_copy(stage.at[sidx[fn]], rows[rn], gs[rn])

                pltpu.sync_copy(rows[r], acc.at[didx[b]], add=True)

                @pl.when(j + _NI < nch)
                def _():
                    pltpu.async_copy(src_hbm.at[wid, j + _NI], sidx[b], fs[b])
                    pltpu.async_copy(dst_hbm.at[wid, j + _NI], didx[b], fs[b])
            return carry

        lax.fori_loop(0, nch // _NI, block, 0)
        plsc.subcore_barrier()
        pltpu.sync_copy(
            acc.at[pl.ds(sid * rpt, rpt)], out_hbm.at[cid, pl.ds(sid * rpt, rpt)]
        )

    return k(ps, src3, dst3)


def _deg_call(dst3, npad):
    w = 16
    _, nch, _ = dst3.shape
    rpt = npad // _NS
    mesh = plsc.VectorSubcoreMesh(core_axis_name="c", subcore_axis_name="s")

    @functools.partial(
        pl.kernel,
        mesh=mesh,
        out_type=jax.ShapeDtypeStruct((_NC, npad, w), jnp.float32),
        scratch_types=[
            pltpu.VMEM((nch, _CH), jnp.int32),
            pltpu.VMEM((_CH, w), jnp.float32),
            pltpu.VMEM((_CH, w), jnp.float32),
            pltpu.VMEM_SHARED((npad, w), jnp.float32),
            pltpu.SemaphoreType.DMA,
        ],
        compiler_params=pltpu.CompilerParams(use_tc_tiling_on_sc=False),
    )
    def k(dst_hbm, out_hbm, didx, ones_v, zrows, acc, sem):
        cid = lax.axis_index("c")
        sid = lax.axis_index("s")
        wid = cid * _NS + sid
        one = jnp.ones((16,), jnp.float32)
        zero = jnp.zeros((16,), jnp.float32)

        pltpu.sync_copy(dst_hbm.at[wid], didx)

        def fill(r, carry):
            ones_v[r, pl.ds(0, 16)] = one
            zrows[r, pl.ds(0, 16)] = zero
            return carry

        lax.fori_loop(0, _CH, fill, 0)
        for z in range(rpt // _CH):
            pltpu.sync_copy(zrows, acc.at[pl.ds(sid * rpt + z * _CH, _CH)])
        plsc.subcore_barrier()

        pltpu.async_copy(ones_v, acc.at[didx.at[0]], sem, add=True)

        def step(i, carry):
            pltpu.async_copy(ones_v, acc.at[didx.at[i + 1]], sem, add=True)
            pltpu.make_async_copy(ones_v, acc.at[didx.at[i]], sem).wait()
            return carry

        lax.fori_loop(0, nch - 1, step, 0)
        pltpu.make_async_copy(ones_v, acc.at[didx.at[nch - 1]], sem).wait()
        plsc.subcore_barrier()
        pltpu.sync_copy(
            acc.at[pl.ds(sid * rpt, rpt)], out_hbm.at[cid, pl.ds(sid * rpt, rpt)]
        )

    return k(dst3)


def _tc1_call(x, degp, w0r, w00, wfa, wf0, bfr):
    n, d = x.shape
    hid = w0r.shape[1]
    emb = wfa.shape[1]
    g = n // _BLK

    def body(x_ref, dp_ref, w0r_ref, w00_ref, wfa_ref, wf0_ref, bf_ref, p1_ref, oa_ref):
        deg = dp_ref[0, :, 0:1] + dp_ref[1, :, 0:1]
        xb = x_ref[...]
        p1_ref[...] = (jnp.dot(xb, w0r_ref[...]) + deg * w00_ref[...]).astype(
            jnp.bfloat16
        )
        oa_ref[...] = jnp.dot(xb, wfa_ref[...]) + deg * wf0_ref[...] + bf_ref[...]

    return pl.pallas_call(
        body,
        grid=(g,),
        in_specs=[
            pl.BlockSpec((_BLK, d), lambda i: (i, 0)),
            pl.BlockSpec((_NC, _BLK, 16), lambda i: (0, i, 0)),
            pl.BlockSpec((d, hid), lambda i: (0, 0)),
            pl.BlockSpec((1, hid), lambda i: (0, 0)),
            pl.BlockSpec((d, emb), lambda i: (0, 0)),
            pl.BlockSpec((1, emb), lambda i: (0, 0)),
            pl.BlockSpec((1, emb), lambda i: (0, 0)),
        ],
        out_specs=[
            pl.BlockSpec((_BLK, hid), lambda i: (i, 0)),
            pl.BlockSpec((_BLK, emb), lambda i: (i, 0)),
        ],
        out_shape=[
            jax.ShapeDtypeStruct((n, hid), jnp.bfloat16),
            jax.ShapeDtypeStruct((n, emb), jnp.float32),
        ],
    )(x, degp, w0r, w00, wfa, wf0, bfr)


def _tc2_call(s1p, degp, p1, b0r, w1, wfb, oa):
    n, hid = p1.shape
    emb = w1.shape[1]
    g = n // _BLK

    def body(s_ref, dp_ref, p1_ref, b0_ref, w1_ref, wfb_ref, oa_ref, p2_ref, o2_ref):
        deg = dp_ref[0, :, 0:1] + dp_ref[1, :, 0:1]
        degc = jnp.maximum(deg, 1.0)
        pos = deg > 0.0
        s = s_ref[0].astype(jnp.float32) + s_ref[1].astype(jnp.float32)
        agg = jnp.where(pos, s / degc, p1_ref[...].astype(jnp.float32))
        h2 = jnp.maximum(agg + b0_ref[...], 0.0)
        p2_ref[...] = jnp.dot(h2, w1_ref[...]).astype(jnp.bfloat16)
        o2_ref[...] = oa_ref[...] + jnp.dot(h2, wfb_ref[...])

    return pl.pallas_call(
        body,
        grid=(g,),
        in_specs=[
            pl.BlockSpec((_NC, _BLK, hid), lambda i: (0, i, 0)),
            pl.BlockSpec((_NC, _BLK, 16), lambda i: (0, i, 0)),
            pl.BlockSpec((_BLK, hid), lambda i: (i, 0)),
            pl.BlockSpec((1, hid), lambda i: (0, 0)),
            pl.BlockSpec((hid, emb), lambda i: (0, 0)),
            pl.BlockSpec((hid, emb), lambda i: (0, 0)),
            pl.BlockSpec((_BLK, emb), lambda i: (i, 0)),
        ],
        out_specs=[
            pl.BlockSpec((_BLK, emb), lambda i: (i, 0)),
            pl.BlockSpec((_BLK, emb), lambda i: (i, 0)),
        ],
        out_shape=[
            jax.ShapeDtypeStruct((n, emb), jnp.bfloat16),
            jax.ShapeDtypeStruct((n, emb), jnp.float32),
        ],
    )(s1p, degp, p1, b0r, w1, wfb, oa)


def _tc3_call(s2p, degp, p2, b1r, wfc, o2):
    n, emb = p2.shape
    g = n // _BLK

    def body(s_ref, dp_ref, p2_ref, b1_ref, wfc_ref, o2_ref, out_ref):
        deg = dp_ref[0, :, 0:1] + dp_ref[1, :, 0:1]
        s = s_ref[0].astype(jnp.float32) + s_ref[1].astype(jnp.float32)
        mean = s / jnp.maximum(deg, 1.0)
        agg = jnp.where(deg > 0.0, mean, p2_ref[...].astype(jnp.float32))
        h3 = jnp.maximum(agg + b1_ref[...], 0.0)
        out_ref[...] = o2_ref[...] + jnp.dot(h3, wfc_ref[...])

    return pl.pallas_call(
        body,
        grid=(g,),
        in_specs=[
            pl.BlockSpec((_NC, _BLK, emb), lambda i: (0, i, 0)),
            pl.BlockSpec((_NC, _BLK, 16), lambda i: (0, i, 0)),
            pl.BlockSpec((_BLK, emb), lambda i: (i, 0)),
            pl.BlockSpec((1, emb), lambda i: (0, 0)),
            pl.BlockSpec((emb, emb), lambda i: (0, 0)),
            pl.BlockSpec((_BLK, emb), lambda i: (i, 0)),
        ],
        out_specs=pl.BlockSpec((_BLK, emb), lambda i: (i, 0)),
        out_shape=jax.ShapeDtypeStruct((n, emb), jnp.float32),
    )(s2p, degp, p2, b1r, wfc, o2)


def kernel(x, edge_index, W0, b0, W1, b1, Wf, bf):
    n, d = x.shape
    e = edge_index.shape[1]
    hid = W0.shape[1]
    emb = W1.shape[1]

    grain = _NS * _CH
    npad = -(-(n + 1) // grain) * grain
    egrain = _NW * _CH * _NI
    epad = -(-e // egrain) * egrain
    nch = epad // (_NW * _CH)

    pad = epad - e
    spare = npad - n
    pad_src = jnp.zeros((pad,), jnp.int32)
    pad_dst = jnp.arange(pad, dtype=jnp.int32) % spare + n
    srcp = jnp.concatenate([edge_index[0], pad_src])
    dstp = jnp.concatenate([edge_index[1], pad_dst])
    src3 = srcp.reshape(_NW, nch, _CH)
    dst3 = dstp.reshape(_NW, nch, _CH)

    degp = _deg_call(dst3, npad)

    p1, oa = _tc1_call(
        x,
        degp,
        W0[1:],
        W0[0:1],
        Wf[1 : d + 1],
        Wf[0:1],
        bf.reshape(1, emb),
    )

    s1p = _seg_sum_call(p1, src3, dst3, npad, split_cols=False)
    p2, o2 = _tc2_call(
        s1p, degp, p1, b0.reshape(1, hid), W1, Wf[d + 1 : d + 1 + hid], oa
    )

    s2p = _seg_sum_call(p2, src3, dst3, npad, split_cols=False)
    out = _tc3_call(s2p, degp, p2, b1.reshape(1, emb), Wf[d + 1 + hid :], o2)
    return out
